# baseline (speedup 1.0000x reference)
import functools
import math

import jax
import jax.numpy as jnp
from jax import lax
from jax.experimental import pallas as pl
from jax.experimental.pallas import tpu as pltpu
from jax.experimental.pallas import tpu_sc as plsc

N = 10000
E = 320000
D_IN = 128
D_EDGE = 16
H = 128
EMB = 128
L = 5
G = 64

_SC_INFO = plsc.get_sparse_core_info()
NC = _SC_INFO.num_cores
NS = _SC_INFO.num_subcores
NW = NC * NS
EPW = E // NW
CH = 40
NCHUNK = EPW // CH
N_PAD = 10240
RPT = N_PAD // NS
ZR = 128

_SQRT2 = math.sqrt(2.0)
_HI = lax.Precision.HIGHEST


_sc_mesh = plsc.VectorSubcoreMesh(core_axis_name="c", subcore_axis_name="s")

NBUF = 2
NGROUP = NCHUNK // NBUF


@functools.partial(
    pl.kernel,
    out_type=jax.ShapeDtypeStruct((NC, N_PAD, H), jnp.float32),
    mesh=_sc_mesh,
    scratch_types=[
        pltpu.VMEM((EPW,), jnp.int32),
        pltpu.VMEM((NBUF, CH), jnp.int32),
        pltpu.VMEM((NBUF, CH, H), jnp.float32),
        pltpu.VMEM((NBUF, CH, H), jnp.float32),
        pltpu.VMEM_SHARED((N_PAD, H), jnp.float32),
    ] + [pltpu.SemaphoreType.DMA] * (2 * NBUF),
)
def _sc_edge_aggr(xn_hbm, e_hbm, src_hbm, dst_hbm, out_hbm,
                  srcv, dstv, msgv, xbuf, aggr_sh, *allsems):
    sems = allsems[:NBUF]
    ssems = allsems[NBUF:]
    cid = lax.axis_index("c")
    sid = lax.axis_index("s")
    wid = sid * NC + cid

    pltpu.sync_copy(src_hbm.at[wid], srcv)

    def _zrow(j, carry):
        for k in range(H // 16):
            xbuf[0, j, pl.ds(k * 16, 16)] = jnp.zeros((16,), jnp.float32)
        return carry
    lax.fori_loop(0, CH, _zrow, 0)
    for c in range(RPT // CH):
        pltpu.sync_copy(xbuf.at[0], aggr_sh.at[pl.ds(sid * RPT + c * CH, CH)])
    plsc.subcore_barrier()

    def _load(i, b):
        pltpu.async_copy(e_hbm.at[wid, i], msgv.at[b], sems[b])
        pltpu.async_copy(dst_hbm.at[wid, i], dstv.at[b], sems[b])
        pltpu.async_copy(xn_hbm.at[srcv.at[pl.ds(i * CH, CH)]], xbuf.at[b],
                         sems[b])

    def _start(i, b):
        pltpu.make_async_copy(msgv.at[b], aggr_sh.at[dstv.at[b]],
                              ssems[b]).wait()
        _load(i, b)

    def _finish(i, b):
        pltpu.make_async_copy(e_hbm.at[wid, i], msgv.at[b], sems[b]).wait()
        pltpu.make_async_copy(dst_hbm.at[wid, i], dstv.at[b], sems[b]).wait()
        pltpu.make_async_copy(xn_hbm.at[srcv.at[pl.ds(i * CH, CH)]],
                              xbuf.at[b], sems[b]).wait()

        @plsc.parallel_loop(0, CH, unroll=2)
        def _row(j):
            for k in range(H // 16):
                sl = pl.ds(k * 16, 16)
                msgv[b, j, sl] = jnp.maximum(msgv[b, j, sl] + xbuf[b, j, sl],
                                             0.0)
        pltpu.async_copy(msgv.at[b], aggr_sh.at[dstv.at[b]], ssems[b],
                         add=True)

    for b in range(NBUF):
        _load(b, b)

    def _group(g, carry):
        i0 = g * NBUF
        for b in range(NBUF):
            _finish(i0 + b, b)
        for b in range(NBUF):
            _start(i0 + b + NBUF, b)
        return carry
    lax.fori_loop(0, NGROUP - 1, _group, 0)
    for b in range(NBUF):
        _finish((NGROUP - 1) * NBUF + b, b)
    for b in range(NBUF):
        pltpu.make_async_copy(msgv.at[b], aggr_sh.at[dstv.at[b]],
                              ssems[b]).wait()

    plsc.subcore_barrier()
    pltpu.sync_copy(aggr_sh.at[pl.ds(sid * RPT, RPT)],
                    out_hbm.at[cid, pl.ds(sid * RPT, RPT)])


RB = 2000
NRB = N // RB


def _accum_stats(t, st_ref):
    @pl.when(pl.program_id(0) == 0)
    def _():
        st_ref[...] = jnp.zeros_like(st_ref)
    st_ref[...] += jnp.concatenate(
        [jnp.sum(t, axis=0, keepdims=True),
         jnp.sum(t * t, axis=0, keepdims=True)], axis=0)


def _mean_var(st_ref):
    m = st_ref[0:1, :] / N
    v = st_ref[1:2, :] / N - m * m
    return m, v


def _k0_body(x_ref, Wi_ref, bi_ref, h_ref, st_ref):
    h = jnp.dot(x_ref[...], Wi_ref[...], preferred_element_type=jnp.float32,
                precision=_HI) + bi_ref[...]
    h_ref[...] = h
    _accum_stats(h, st_ref)


def _call_k0(x, W_in, b_in):
    return pl.pallas_call(
        _k0_body,
        grid=(NRB,),
        in_specs=[
            pl.BlockSpec((RB, D_IN), lambda i: (i, 0)),
            pl.BlockSpec((D_IN, H), lambda i: (0, 0)),
            pl.BlockSpec((1, H), lambda i: (0, 0)),
        ],
        out_specs=(pl.BlockSpec((RB, H), lambda i: (i, 0)),
                   pl.BlockSpec((2, H), lambda i: (0, 0))),
        out_shape=(jax.ShapeDtypeStruct((N, H), jnp.float32),
                   jax.ShapeDtypeStruct((2, H), jnp.float32)),
    )(x, W_in, b_in)


def _ka_body(h_ref, st_ref, g_ref, b_ref, xn_ref):
    m, v = _mean_var(st_ref)
    xn_ref[...] = (g_ref[...] * (h_ref[...] - m) / jnp.sqrt(v + 1e-5)
                   + b_ref[...])


def _call_ka(h, st, g, b):
    return pl.pallas_call(
        _ka_body,
        grid=(NRB,),
        in_specs=[
            pl.BlockSpec((RB, H), lambda i: (i, 0)),
            pl.BlockSpec((2, H), lambda i: (0, 0)),
            pl.BlockSpec((1, H), lambda i: (0, 0)),
            pl.BlockSpec((1, H), lambda i: (0, 0)),
        ],
        out_specs=pl.BlockSpec((RB, H), lambda i: (i, 0)),
        out_shape=jax.ShapeDtypeStruct((N, H), jnp.float32),
    )(h, st, g, b)


_EBLK = 4000


def _ke_body(ea_ref, We_ref, be_ref, e_ref):
    e_ref[...] = jnp.dot(ea_ref[...], We_ref[...],
                         preferred_element_type=jnp.float32,
                         precision=_HI) + be_ref[...]


def _call_ke(edge_attr, We_l, be_l):
    return pl.pallas_call(
        _ke_body,
        grid=(E // _EBLK,),
        in_specs=[
            pl.BlockSpec((_EBLK, D_EDGE), lambda i: (i, 0)),
            pl.BlockSpec((D_EDGE, H), lambda i: (0, 0)),
            pl.BlockSpec((1, H), lambda i: (0, 0)),
        ],
        out_specs=pl.BlockSpec((_EBLK, H), lambda i: (i, 0)),
        out_shape=jax.ShapeDtypeStruct((E, H), jnp.float32),
    )(edge_attr, We_l, be_l)


def _kb1_body(xn_ref, p_ref, eps_ref, W1_ref, b1_ref, t_ref, st_ref):
    z = (1.0 + eps_ref[0, 0]) * xn_ref[...] + p_ref[0] + p_ref[1]
    t = jnp.dot(z, W1_ref[...], preferred_element_type=jnp.float32,
                precision=_HI) + b1_ref[...]
    t_ref[...] = t
    _accum_stats(t, st_ref)


def _call_kb1(xn, parts, eps_l, W1_l, b1_l):
    return pl.pallas_call(
        _kb1_body,
        grid=(NRB,),
        in_specs=[
            pl.BlockSpec((RB, H), lambda i: (i, 0)),
            pl.BlockSpec((2, RB, H), lambda i: (0, i, 0)),
            pl.BlockSpec((1, 1), lambda i: (0, 0)),
            pl.BlockSpec((H, 2 * H), lambda i: (0, 0)),
            pl.BlockSpec((1, 2 * H), lambda i: (0, 0)),
        ],
        out_specs=(pl.BlockSpec((RB, 2 * H), lambda i: (i, 0)),
                   pl.BlockSpec((2, 2 * H), lambda i: (0, 0))),
        out_shape=(jax.ShapeDtypeStruct((N, 2 * H), jnp.float32),
                   jax.ShapeDtypeStruct((2, 2 * H), jnp.float32)),
    )(xn, parts, eps_l, W1_l, b1_l)


def _kb2_body(t_ref, st_ref, g2_ref, b2_ref, W2_ref, bb_ref, h_ref,
              hn_ref, sth_ref):
    m, v = _mean_var(st_ref)
    tn = g2_ref[...] * (t_ref[...] - m) / jnp.sqrt(v + 1e-5) + b2_ref[...]
    tg = 0.5 * tn * (1.0 + lax.erf(tn / _SQRT2))
    z2 = jnp.dot(tg, W2_ref[...], preferred_element_type=jnp.float32,
                 precision=_HI) + bb_ref[...]
    hn = h_ref[...] + z2
    hn_ref[...] = hn
    _accum_stats(hn, sth_ref)


def _call_kb2(t, st, g2, b2, W2, bb, h):
    return pl.pallas_call(
        _kb2_body,
        grid=(NRB,),
        in_specs=[
            pl.BlockSpec((RB, 2 * H), lambda i: (i, 0)),
            pl.BlockSpec((2, 2 * H), lambda i: (0, 0)),
            pl.BlockSpec((1, 2 * H), lambda i: (0, 0)),
            pl.BlockSpec((1, 2 * H), lambda i: (0, 0)),
            pl.BlockSpec((2 * H, H), lambda i: (0, 0)),
            pl.BlockSpec((1, H), lambda i: (0, 0)),
            pl.BlockSpec((RB, H), lambda i: (i, 0)),
        ],
        out_specs=(pl.BlockSpec((RB, H), lambda i: (i, 0)),
                   pl.BlockSpec((2, H), lambda i: (0, 0))),
        out_shape=(jax.ShapeDtypeStruct((N, H), jnp.float32),
                   jax.ShapeDtypeStruct((2, H), jnp.float32)),
    )(t, st, g2, b2, W2, bb, h)


def _kc_body(o0, o1, o2, o3, o4, Wj_ref, bj_ref, Wa_ref, ba_ref,
             batch_ref, pooled_ref):
    xcat = jnp.concatenate([o0[...], o1[...], o2[...], o3[...], o4[...]],
                           axis=1)
    xf = jnp.dot(xcat, Wj_ref[...], preferred_element_type=jnp.float32,
                 precision=_HI) + bj_ref[...]
    logit = jnp.sum(xf * Wa_ref[...], axis=1, keepdims=True) + ba_ref[...]
    att = 1.0 / (1.0 + jnp.exp(-logit))
    w = xf * att
    gids = lax.broadcasted_iota(jnp.int32, (1, G), 1)
    onehot = (batch_ref[...] == gids).astype(jnp.float32)
    pool = lax.dot_general(onehot, w, (((0,), (0,)), ((), ())),
                           preferred_element_type=jnp.float32,
                           precision=_HI)

    @pl.when(pl.program_id(0) == 0)
    def _():
        pooled_ref[...] = jnp.zeros_like(pooled_ref)
    pooled_ref[...] += pool


def _call_kc(outs, W_jump, b_jump, W_att, b_att, batch2):
    return pl.pallas_call(
        _kc_body,
        grid=(NRB,),
        in_specs=[pl.BlockSpec((RB, H), lambda i: (i, 0))] * 5 + [
            pl.BlockSpec((L * H, H), lambda i: (0, 0)),
            pl.BlockSpec((1, H), lambda i: (0, 0)),
            pl.BlockSpec((1, H), lambda i: (0, 0)),
            pl.BlockSpec((1, 1), lambda i: (0, 0)),
            pl.BlockSpec((RB, 1), lambda i: (i, 0)),
        ],
        out_specs=pl.BlockSpec((G, EMB), lambda i: (0, 0)),
        out_shape=jax.ShapeDtypeStruct((G, EMB), jnp.float32),
    )(*outs, W_jump, b_jump, W_att, b_att, batch2)


def _kd_body(pooled_ref, Wo1_ref, bo1_ref, Wo2_ref, bo2_ref, out_ref):
    t = jnp.dot(pooled_ref[...], Wo1_ref[...],
                preferred_element_type=jnp.float32, precision=_HI) + bo1_ref[...]
    t = jnp.maximum(t, 0.0)
    emb = jnp.dot(t, Wo2_ref[...], preferred_element_type=jnp.float32,
                  precision=_HI) + bo2_ref[...]
    nrm = jnp.maximum(jnp.sqrt(jnp.sum(emb * emb, axis=1, keepdims=True)),
                      1e-12)
    out_ref[...] = emb / nrm


def _call_kd(pooled, W_o1, b_o1, W_o2, b_o2):
    return pl.pallas_call(
        _kd_body,
        out_shape=jax.ShapeDtypeStruct((G, EMB), jnp.float32),
    )(pooled, W_o1, b_o1, W_o2, b_o2)


def kernel(x, edge_index, edge_attr, batch, W_in, b_in, bn_g, bn_b, eps,
           We, be, W1, b1, bn2_g, bn2_b, W2, b2, W_jump, b_jump, W_att,
           b_att, W_o1, b_o1, W_o2, b_o2):
    src = edge_index[0].reshape(NW, EPW)
    dst = edge_index[1].reshape(NW, NCHUNK, CH)
    batch2 = batch.reshape(N, 1)

    h, st_h = _call_k0(x, W_in, b_in.reshape(1, H))

    es = [_call_ke(edge_attr, We[l], be[l].reshape(1, H)).reshape(
        NW, NCHUNK, CH, H) for l in range(L)]

    outs = []
    for l in range(L):
        xn = _call_ka(h, st_h, bn_g[l].reshape(1, H), bn_b[l].reshape(1, H))
        parts = _sc_edge_aggr(xn, es[l], src, dst)[:, :N, :]
        t, st_t = _call_kb1(xn, parts, eps[l].reshape(1, 1), W1[l],
                            b1[l].reshape(1, 2 * H))
        h, st_h = _call_kb2(t, st_t, bn2_g[l].reshape(1, 2 * H),
                            bn2_b[l].reshape(1, 2 * H), W2[l],
                            b2[l].reshape(1, H), h)
        outs.append(h)

    pooled = _call_kc(outs, W_jump, b_jump.reshape(1, H),
                      W_att.reshape(1, H), b_att.reshape(1, 1), batch2)
    return _call_kd(pooled, W_o1, b_o1.reshape(1, EMB), W_o2,
                    b_o2.reshape(1, EMB))


# speedup vs baseline: 1.0710x; 1.0710x over previous
import functools
import math

import jax
import jax.numpy as jnp
from jax import lax
from jax.experimental import pallas as pl
from jax.experimental.pallas import tpu as pltpu
from jax.experimental.pallas import tpu_sc as plsc

N = 10000
E = 320000
D_IN = 128
D_EDGE = 16
H = 128
EMB = 128
L = 5
G = 64

_SC_INFO = plsc.get_sparse_core_info()
NC = _SC_INFO.num_cores
NS = _SC_INFO.num_subcores
NW = NC * NS
EPW = E // NW
CH = 40
NCHUNK = EPW // CH
N_PAD = 10240
RPT = N_PAD // NS
ZR = 128

_SQRT2 = math.sqrt(2.0)
_HI = lax.Precision.HIGHEST


_sc_mesh = plsc.VectorSubcoreMesh(core_axis_name="c", subcore_axis_name="s")

NBUF = 2
NGROUP = NCHUNK // NBUF


@functools.partial(
    pl.kernel,
    out_type=jax.ShapeDtypeStruct((NC, N_PAD, H), jnp.float32),
    mesh=_sc_mesh,
    scratch_types=[
        pltpu.VMEM((EPW,), jnp.int32),
        pltpu.VMEM((NBUF, CH), jnp.int32),
        pltpu.VMEM((NBUF, CH, H), jnp.float32),
        pltpu.VMEM((NBUF, CH, H), jnp.float32),
        pltpu.VMEM_SHARED((N_PAD, H), jnp.float32),
    ] + [pltpu.SemaphoreType.DMA] * (2 * NBUF),
)
def _sc_edge_aggr(xn_hbm, e_hbm, src_hbm, dst_hbm, out_hbm,
                  srcv, dstv, msgv, xbuf, aggr_sh, *allsems):
    sems = allsems[:NBUF]
    ssems = allsems[NBUF:]
    cid = lax.axis_index("c")
    sid = lax.axis_index("s")
    wid = sid * NC + cid

    pltpu.sync_copy(src_hbm.at[wid], srcv)

    def _zrow(j, carry):
        for k in range(H // 16):
            xbuf[0, j, pl.ds(k * 16, 16)] = jnp.zeros((16,), jnp.float32)
        return carry
    lax.fori_loop(0, CH, _zrow, 0)
    for c in range(RPT // CH):
        pltpu.sync_copy(xbuf.at[0], aggr_sh.at[pl.ds(sid * RPT + c * CH, CH)])
    plsc.subcore_barrier()

    def _load(i, b):
        pltpu.async_copy(e_hbm.at[wid, i], msgv.at[b], sems[b])
        pltpu.async_copy(dst_hbm.at[wid, i], dstv.at[b], sems[b])
        pltpu.async_copy(xn_hbm.at[srcv.at[pl.ds(i * CH, CH)]], xbuf.at[b],
                         sems[b])

    def _start(i, b):
        pltpu.make_async_copy(msgv.at[b], aggr_sh.at[dstv.at[b]],
                              ssems[b]).wait()
        _load(i, b)

    def _finish(i, b):
        pltpu.make_async_copy(e_hbm.at[wid, i], msgv.at[b], sems[b]).wait()
        pltpu.make_async_copy(dst_hbm.at[wid, i], dstv.at[b], sems[b]).wait()
        pltpu.make_async_copy(xn_hbm.at[srcv.at[pl.ds(i * CH, CH)]],
                              xbuf.at[b], sems[b]).wait()

        @plsc.parallel_loop(0, CH, unroll=2)
        def _row(j):
            for k in range(H // 16):
                sl = pl.ds(k * 16, 16)
                msgv[b, j, sl] = jnp.maximum(msgv[b, j, sl] + xbuf[b, j, sl],
                                             0.0)
        pltpu.async_copy(msgv.at[b], aggr_sh.at[dstv.at[b]], ssems[b],
                         add=True)

    for b in range(NBUF):
        _load(b, b)

    def _group(g, carry):
        i0 = g * NBUF
        for b in range(NBUF):
            _finish(i0 + b, b)
            _start(i0 + b + NBUF, b)
        return carry
    lax.fori_loop(0, NGROUP - 1, _group, 0)
    for b in range(NBUF):
        _finish((NGROUP - 1) * NBUF + b, b)
    for b in range(NBUF):
        pltpu.make_async_copy(msgv.at[b], aggr_sh.at[dstv.at[b]],
                              ssems[b]).wait()

    plsc.subcore_barrier()
    pltpu.sync_copy(aggr_sh.at[pl.ds(sid * RPT, RPT)],
                    out_hbm.at[cid, pl.ds(sid * RPT, RPT)])


RB = 2000
NRB = N // RB


def _accum_stats(t, st_ref):
    @pl.when(pl.program_id(0) == 0)
    def _():
        st_ref[...] = jnp.zeros_like(st_ref)
    st_ref[...] += jnp.concatenate(
        [jnp.sum(t, axis=0, keepdims=True),
         jnp.sum(t * t, axis=0, keepdims=True)], axis=0)


def _mean_var(st_ref):
    m = st_ref[0:1, :] / N
    v = st_ref[1:2, :] / N - m * m
    return m, v


def _k0_body(x_ref, Wi_ref, bi_ref, h_ref, st_ref):
    h = jnp.dot(x_ref[...], Wi_ref[...], preferred_element_type=jnp.float32,
                precision=_HI) + bi_ref[...]
    h_ref[...] = h
    _accum_stats(h, st_ref)


def _call_k0(x, W_in, b_in):
    return pl.pallas_call(
        _k0_body,
        grid=(NRB,),
        in_specs=[
            pl.BlockSpec((RB, D_IN), lambda i: (i, 0)),
            pl.BlockSpec((D_IN, H), lambda i: (0, 0)),
            pl.BlockSpec((1, H), lambda i: (0, 0)),
        ],
        out_specs=(pl.BlockSpec((RB, H), lambda i: (i, 0)),
                   pl.BlockSpec((2, H), lambda i: (0, 0))),
        out_shape=(jax.ShapeDtypeStruct((N, H), jnp.float32),
                   jax.ShapeDtypeStruct((2, H), jnp.float32)),
    )(x, W_in, b_in)


def _ka_body(h_ref, st_ref, g_ref, b_ref, xn_ref):
    m, v = _mean_var(st_ref)
    xn_ref[...] = (g_ref[...] * (h_ref[...] - m) / jnp.sqrt(v + 1e-5)
                   + b_ref[...])


def _call_ka(h, st, g, b):
    return pl.pallas_call(
        _ka_body,
        grid=(NRB,),
        in_specs=[
            pl.BlockSpec((RB, H), lambda i: (i, 0)),
            pl.BlockSpec((2, H), lambda i: (0, 0)),
            pl.BlockSpec((1, H), lambda i: (0, 0)),
            pl.BlockSpec((1, H), lambda i: (0, 0)),
        ],
        out_specs=pl.BlockSpec((RB, H), lambda i: (i, 0)),
        out_shape=jax.ShapeDtypeStruct((N, H), jnp.float32),
    )(h, st, g, b)


_EBLK = 4000


def _ke_body(ea_ref, We_ref, be_ref, e_ref):
    e_ref[...] = jnp.dot(ea_ref[...], We_ref[...],
                         preferred_element_type=jnp.float32,
                         precision=_HI) + be_ref[...]


def _call_ke(edge_attr, We_l, be_l):
    return pl.pallas_call(
        _ke_body,
        grid=(E // _EBLK,),
        in_specs=[
            pl.BlockSpec((_EBLK, D_EDGE), lambda i: (i, 0)),
            pl.BlockSpec((D_EDGE, H), lambda i: (0, 0)),
            pl.BlockSpec((1, H), lambda i: (0, 0)),
        ],
        out_specs=pl.BlockSpec((_EBLK, H), lambda i: (i, 0)),
        out_shape=jax.ShapeDtypeStruct((E, H), jnp.float32),
    )(edge_attr, We_l, be_l)


def _kb1_body(xn_ref, p_ref, eps_ref, W1_ref, b1_ref, t_ref, st_ref):
    z = (1.0 + eps_ref[0, 0]) * xn_ref[...] + p_ref[0] + p_ref[1]
    t = jnp.dot(z, W1_ref[...], preferred_element_type=jnp.float32,
                precision=_HI) + b1_ref[...]
    t_ref[...] = t
    _accum_stats(t, st_ref)


def _call_kb1(xn, parts, eps_l, W1_l, b1_l):
    return pl.pallas_call(
        _kb1_body,
        grid=(NRB,),
        in_specs=[
            pl.BlockSpec((RB, H), lambda i: (i, 0)),
            pl.BlockSpec((2, RB, H), lambda i: (0, i, 0)),
            pl.BlockSpec((1, 1), lambda i: (0, 0)),
            pl.BlockSpec((H, 2 * H), lambda i: (0, 0)),
            pl.BlockSpec((1, 2 * H), lambda i: (0, 0)),
        ],
        out_specs=(pl.BlockSpec((RB, 2 * H), lambda i: (i, 0)),
                   pl.BlockSpec((2, 2 * H), lambda i: (0, 0))),
        out_shape=(jax.ShapeDtypeStruct((N, 2 * H), jnp.float32),
                   jax.ShapeDtypeStruct((2, 2 * H), jnp.float32)),
    )(xn, parts, eps_l, W1_l, b1_l)


def _kb2_body(t_ref, st_ref, g2_ref, b2_ref, W2_ref, bb_ref, h_ref,
              hn_ref, sth_ref):
    m, v = _mean_var(st_ref)
    tn = g2_ref[...] * (t_ref[...] - m) / jnp.sqrt(v + 1e-5) + b2_ref[...]
    tg = 0.5 * tn * (1.0 + lax.erf(tn / _SQRT2))
    z2 = jnp.dot(tg, W2_ref[...], preferred_element_type=jnp.float32,
                 precision=_HI) + bb_ref[...]
    hn = h_ref[...] + z2
    hn_ref[...] = hn
    _accum_stats(hn, sth_ref)


def _call_kb2(t, st, g2, b2, W2, bb, h):
    return pl.pallas_call(
        _kb2_body,
        grid=(NRB,),
        in_specs=[
            pl.BlockSpec((RB, 2 * H), lambda i: (i, 0)),
            pl.BlockSpec((2, 2 * H), lambda i: (0, 0)),
            pl.BlockSpec((1, 2 * H), lambda i: (0, 0)),
            pl.BlockSpec((1, 2 * H), lambda i: (0, 0)),
            pl.BlockSpec((2 * H, H), lambda i: (0, 0)),
            pl.BlockSpec((1, H), lambda i: (0, 0)),
            pl.BlockSpec((RB, H), lambda i: (i, 0)),
        ],
        out_specs=(pl.BlockSpec((RB, H), lambda i: (i, 0)),
                   pl.BlockSpec((2, H), lambda i: (0, 0))),
        out_shape=(jax.ShapeDtypeStruct((N, H), jnp.float32),
                   jax.ShapeDtypeStruct((2, H), jnp.float32)),
    )(t, st, g2, b2, W2, bb, h)


def _kc_body(o0, o1, o2, o3, o4, Wj_ref, bj_ref, Wa_ref, ba_ref,
             batch_ref, pooled_ref):
    xcat = jnp.concatenate([o0[...], o1[...], o2[...], o3[...], o4[...]],
                           axis=1)
    xf = jnp.dot(xcat, Wj_ref[...], preferred_element_type=jnp.float32,
                 precision=_HI) + bj_ref[...]
    logit = jnp.sum(xf * Wa_ref[...], axis=1, keepdims=True) + ba_ref[...]
    att = 1.0 / (1.0 + jnp.exp(-logit))
    w = xf * att
    gids = lax.broadcasted_iota(jnp.int32, (1, G), 1)
    onehot = (batch_ref[...] == gids).astype(jnp.float32)
    pool = lax.dot_general(onehot, w, (((0,), (0,)), ((), ())),
                           preferred_element_type=jnp.float32,
                           precision=_HI)

    @pl.when(pl.program_id(0) == 0)
    def _():
        pooled_ref[...] = jnp.zeros_like(pooled_ref)
    pooled_ref[...] += pool


def _call_kc(outs, W_jump, b_jump, W_att, b_att, batch2):
    return pl.pallas_call(
        _kc_body,
        grid=(NRB,),
        in_specs=[pl.BlockSpec((RB, H), lambda i: (i, 0))] * 5 + [
            pl.BlockSpec((L * H, H), lambda i: (0, 0)),
            pl.BlockSpec((1, H), lambda i: (0, 0)),
            pl.BlockSpec((1, H), lambda i: (0, 0)),
            pl.BlockSpec((1, 1), lambda i: (0, 0)),
            pl.BlockSpec((RB, 1), lambda i: (i, 0)),
        ],
        out_specs=pl.BlockSpec((G, EMB), lambda i: (0, 0)),
        out_shape=jax.ShapeDtypeStruct((G, EMB), jnp.float32),
    )(*outs, W_jump, b_jump, W_att, b_att, batch2)


def _kd_body(pooled_ref, Wo1_ref, bo1_ref, Wo2_ref, bo2_ref, out_ref):
    t = jnp.dot(pooled_ref[...], Wo1_ref[...],
                preferred_element_type=jnp.float32, precision=_HI) + bo1_ref[...]
    t = jnp.maximum(t, 0.0)
    emb = jnp.dot(t, Wo2_ref[...], preferred_element_type=jnp.float32,
                  precision=_HI) + bo2_ref[...]
    nrm = jnp.maximum(jnp.sqrt(jnp.sum(emb * emb, axis=1, keepdims=True)),
                      1e-12)
    out_ref[...] = emb / nrm


def _call_kd(pooled, W_o1, b_o1, W_o2, b_o2):
    return pl.pallas_call(
        _kd_body,
        out_shape=jax.ShapeDtypeStruct((G, EMB), jnp.float32),
    )(pooled, W_o1, b_o1, W_o2, b_o2)


def kernel(x, edge_index, edge_attr, batch, W_in, b_in, bn_g, bn_b, eps,
           We, be, W1, b1, bn2_g, bn2_b, W2, b2, W_jump, b_jump, W_att,
           b_att, W_o1, b_o1, W_o2, b_o2):
    src = edge_index[0].reshape(NW, EPW)
    dst = edge_index[1].reshape(NW, NCHUNK, CH)
    batch2 = batch.reshape(N, 1)

    h, st_h = _call_k0(x, W_in, b_in.reshape(1, H))

    es = [_call_ke(edge_attr, We[l], be[l].reshape(1, H)).reshape(
        NW, NCHUNK, CH, H) for l in range(L)]

    outs = []
    for l in range(L):
        xn = _call_ka(h, st_h, bn_g[l].reshape(1, H), bn_b[l].reshape(1, H))
        parts = _sc_edge_aggr(xn, es[l], src, dst)[:, :N, :]
        t, st_t = _call_kb1(xn, parts, eps[l].reshape(1, 1), W1[l],
                            b1[l].reshape(1, 2 * H))
        h, st_h = _call_kb2(t, st_t, bn2_g[l].reshape(1, 2 * H),
                            bn2_b[l].reshape(1, 2 * H), W2[l],
                            b2[l].reshape(1, H), h)
        outs.append(h)

    pooled = _call_kc(outs, W_jump, b_jump.reshape(1, H),
                      W_att.reshape(1, H), b_att.reshape(1, 1), batch2)
    return _call_kd(pooled, W_o1, b_o1.reshape(1, EMB), W_o2,
                    b_o2.reshape(1, EMB))


# speedup vs baseline: 1.1465x; 1.0705x over previous
import functools
import math

import jax
import jax.numpy as jnp
from jax import lax
from jax.experimental import pallas as pl
from jax.experimental.pallas import tpu as pltpu
from jax.experimental.pallas import tpu_sc as plsc

N = 10000
E = 320000
D_IN = 128
D_EDGE = 16
H = 128
EMB = 128
L = 5
G = 64

_SC_INFO = plsc.get_sparse_core_info()
NC = _SC_INFO.num_cores
NS = _SC_INFO.num_subcores
NW = NC * NS
E_PAD = 322560
EPW = E_PAD // NW
CH = 72
NCHUNK = EPW // CH
N_PAD = 10112
RPT = N_PAD // NS

_SQRT2 = math.sqrt(2.0)
_HI = lax.Precision.HIGHEST


_sc_mesh = plsc.VectorSubcoreMesh(core_axis_name="c", subcore_axis_name="s")

NBUF = 2
NGROUP = NCHUNK // NBUF


@functools.partial(
    pl.kernel,
    out_type=jax.ShapeDtypeStruct((NC, N_PAD, H), jnp.float32),
    mesh=_sc_mesh,
    scratch_types=[
        pltpu.VMEM((EPW,), jnp.int32),
        pltpu.VMEM((NBUF, CH), jnp.int32),
        pltpu.VMEM((NBUF, CH, H), jnp.float32),
        pltpu.VMEM((NBUF, CH, H), jnp.float32),
        pltpu.VMEM_SHARED((N_PAD, H), jnp.float32),
    ] + [pltpu.SemaphoreType.DMA] * (2 * NBUF),
)
def _sc_edge_aggr(xn_hbm, e_hbm, src_hbm, dst_hbm, out_hbm,
                  srcv, dstv, msgv, xbuf, aggr_sh, *allsems):
    sems = allsems[:NBUF]
    ssems = allsems[NBUF:]
    cid = lax.axis_index("c")
    sid = lax.axis_index("s")
    wid = sid * NC + cid

    pltpu.sync_copy(src_hbm.at[wid], srcv)

    def _zrow(j, carry):
        for k in range(H // 16):
            xbuf[0, j, pl.ds(k * 16, 16)] = jnp.zeros((16,), jnp.float32)
        return carry
    lax.fori_loop(0, 64, _zrow, 0)
    for c in range(9):
        pltpu.sync_copy(xbuf.at[0, pl.ds(0, 64)],
                        aggr_sh.at[pl.ds(sid * RPT + c * 64, 64)])
    pltpu.sync_copy(xbuf.at[0, pl.ds(0, 56)],
                    aggr_sh.at[pl.ds(sid * RPT + 576, 56)])
    plsc.subcore_barrier()

    def _load(i, b):
        pltpu.async_copy(e_hbm.at[wid, i], msgv.at[b], sems[b])
        pltpu.async_copy(dst_hbm.at[wid, i], dstv.at[b], sems[b])
        pltpu.async_copy(xn_hbm.at[srcv.at[pl.ds(i * CH, CH)]], xbuf.at[b],
                         sems[b])

    def _start(i, b):
        pltpu.make_async_copy(msgv.at[b], aggr_sh.at[dstv.at[b]],
                              ssems[b]).wait()
        _load(i, b)

    def _finish(i, b):
        pltpu.make_async_copy(e_hbm.at[wid, i], msgv.at[b], sems[b]).wait()
        pltpu.make_async_copy(dst_hbm.at[wid, i], dstv.at[b], sems[b]).wait()
        pltpu.make_async_copy(xn_hbm.at[srcv.at[pl.ds(i * CH, CH)]],
                              xbuf.at[b], sems[b]).wait()

        @plsc.parallel_loop(0, CH, unroll=2)
        def _row(j):
            for k in range(H // 16):
                sl = pl.ds(k * 16, 16)
                msgv[b, j, sl] = jnp.maximum(msgv[b, j, sl] + xbuf[b, j, sl],
                                             0.0)
        pltpu.async_copy(msgv.at[b], aggr_sh.at[dstv.at[b]], ssems[b],
                         add=True)

    for b in range(NBUF):
        _load(b, b)

    def _group(g, carry):
        i0 = g * NBUF
        for b in range(NBUF):
            _finish(i0 + b, b)
            _start(i0 + b + NBUF, b)
        return carry
    lax.fori_loop(0, NGROUP - 1, _group, 0)
    for b in range(NBUF):
        _finish((NGROUP - 1) * NBUF + b, b)
    for b in range(NBUF):
        pltpu.make_async_copy(msgv.at[b], aggr_sh.at[dstv.at[b]],
                              ssems[b]).wait()

    plsc.subcore_barrier()
    pltpu.sync_copy(aggr_sh.at[pl.ds(sid * RPT, RPT)],
                    out_hbm.at[cid, pl.ds(sid * RPT, RPT)])


RB = 2000
NRB = N // RB


def _accum_stats(t, st_ref):
    @pl.when(pl.program_id(0) == 0)
    def _():
        st_ref[...] = jnp.zeros_like(st_ref)
    st_ref[...] += jnp.concatenate(
        [jnp.sum(t, axis=0, keepdims=True),
         jnp.sum(t * t, axis=0, keepdims=True)], axis=0)


def _mean_var(st_ref):
    m = st_ref[0:1, :] / N
    v = st_ref[1:2, :] / N - m * m
    return m, v


def _k0_body(x_ref, Wi_ref, bi_ref, h_ref, st_ref):
    h = jnp.dot(x_ref[...], Wi_ref[...], preferred_element_type=jnp.float32,
                precision=_HI) + bi_ref[...]
    h_ref[...] = h
    _accum_stats(h, st_ref)


def _call_k0(x, W_in, b_in):
    return pl.pallas_call(
        _k0_body,
        grid=(NRB,),
        in_specs=[
            pl.BlockSpec((RB, D_IN), lambda i: (i, 0)),
            pl.BlockSpec((D_IN, H), lambda i: (0, 0)),
            pl.BlockSpec((1, H), lambda i: (0, 0)),
        ],
        out_specs=(pl.BlockSpec((RB, H), lambda i: (i, 0)),
                   pl.BlockSpec((2, H), lambda i: (0, 0))),
        out_shape=(jax.ShapeDtypeStruct((N, H), jnp.float32),
                   jax.ShapeDtypeStruct((2, H), jnp.float32)),
    )(x, W_in, b_in)


def _ka_body(h_ref, st_ref, g_ref, b_ref, xn_ref):
    m, v = _mean_var(st_ref)
    xn_ref[...] = (g_ref[...] * (h_ref[...] - m) / jnp.sqrt(v + 1e-5)
                   + b_ref[...])


def _call_ka(h, st, g, b):
    return pl.pallas_call(
        _ka_body,
        grid=(NRB,),
        in_specs=[
            pl.BlockSpec((RB, H), lambda i: (i, 0)),
            pl.BlockSpec((2, H), lambda i: (0, 0)),
            pl.BlockSpec((1, H), lambda i: (0, 0)),
            pl.BlockSpec((1, H), lambda i: (0, 0)),
        ],
        out_specs=pl.BlockSpec((RB, H), lambda i: (i, 0)),
        out_shape=jax.ShapeDtypeStruct((N, H), jnp.float32),
    )(h, st, g, b)


_EBLK = 5040


def _ke_body(ea_ref, We_ref, be_ref, e_ref):
    e_ref[...] = jnp.dot(ea_ref[...], We_ref[...],
                         preferred_element_type=jnp.float32,
                         precision=_HI) + be_ref[...]


def _call_ke(edge_attr, We_l, be_l):
    return pl.pallas_call(
        _ke_body,
        grid=(E_PAD // _EBLK,),
        in_specs=[
            pl.BlockSpec((_EBLK, D_EDGE), lambda i: (i, 0)),
            pl.BlockSpec((D_EDGE, H), lambda i: (0, 0)),
            pl.BlockSpec((1, H), lambda i: (0, 0)),
        ],
        out_specs=pl.BlockSpec((_EBLK, H), lambda i: (i, 0)),
        out_shape=jax.ShapeDtypeStruct((E_PAD, H), jnp.float32),
    )(edge_attr, We_l, be_l)


def _kb1_body(xn_ref, p_ref, eps_ref, W1_ref, b1_ref, t_ref, st_ref):
    z = (1.0 + eps_ref[0, 0]) * xn_ref[...] + p_ref[0] + p_ref[1]
    t = jnp.dot(z, W1_ref[...], preferred_element_type=jnp.float32,
                precision=_HI) + b1_ref[...]
    t_ref[...] = t
    _accum_stats(t, st_ref)


def _call_kb1(xn, parts, eps_l, W1_l, b1_l):
    return pl.pallas_call(
        _kb1_body,
        grid=(NRB,),
        in_specs=[
            pl.BlockSpec((RB, H), lambda i: (i, 0)),
            pl.BlockSpec((2, RB, H), lambda i: (0, i, 0)),
            pl.BlockSpec((1, 1), lambda i: (0, 0)),
            pl.BlockSpec((H, 2 * H), lambda i: (0, 0)),
            pl.BlockSpec((1, 2 * H), lambda i: (0, 0)),
        ],
        out_specs=(pl.BlockSpec((RB, 2 * H), lambda i: (i, 0)),
                   pl.BlockSpec((2, 2 * H), lambda i: (0, 0))),
        out_shape=(jax.ShapeDtypeStruct((N, 2 * H), jnp.float32),
                   jax.ShapeDtypeStruct((2, 2 * H), jnp.float32)),
    )(xn, parts, eps_l, W1_l, b1_l)


def _kb2_body(t_ref, st_ref, g2_ref, b2_ref, W2_ref, bb_ref, h_ref,
              hn_ref, sth_ref):
    m, v = _mean_var(st_ref)
    tn = g2_ref[...] * (t_ref[...] - m) / jnp.sqrt(v + 1e-5) + b2_ref[...]
    tg = 0.5 * tn * (1.0 + lax.erf(tn / _SQRT2))
    z2 = jnp.dot(tg, W2_ref[...], preferred_element_type=jnp.float32,
                 precision=_HI) + bb_ref[...]
    hn = h_ref[...] + z2
    hn_ref[...] = hn
    _accum_stats(hn, sth_ref)


def _call_kb2(t, st, g2, b2, W2, bb, h):
    return pl.pallas_call(
        _kb2_body,
        grid=(NRB,),
        in_specs=[
            pl.BlockSpec((RB, 2 * H), lambda i: (i, 0)),
            pl.BlockSpec((2, 2 * H), lambda i: (0, 0)),
            pl.BlockSpec((1, 2 * H), lambda i: (0, 0)),
            pl.BlockSpec((1, 2 * H), lambda i: (0, 0)),
            pl.BlockSpec((2 * H, H), lambda i: (0, 0)),
            pl.BlockSpec((1, H), lambda i: (0, 0)),
            pl.BlockSpec((RB, H), lambda i: (i, 0)),
        ],
        out_specs=(pl.BlockSpec((RB, H), lambda i: (i, 0)),
                   pl.BlockSpec((2, H), lambda i: (0, 0))),
        out_shape=(jax.ShapeDtypeStruct((N, H), jnp.float32),
                   jax.ShapeDtypeStruct((2, H), jnp.float32)),
    )(t, st, g2, b2, W2, bb, h)


def _kc_body(o0, o1, o2, o3, o4, Wj_ref, bj_ref, Wa_ref, ba_ref,
             batch_ref, pooled_ref):
    xcat = jnp.concatenate([o0[...], o1[...], o2[...], o3[...], o4[...]],
                           axis=1)
    xf = jnp.dot(xcat, Wj_ref[...], preferred_element_type=jnp.float32,
                 precision=_HI) + bj_ref[...]
    logit = jnp.sum(xf * Wa_ref[...], axis=1, keepdims=True) + ba_ref[...]
    att = 1.0 / (1.0 + jnp.exp(-logit))
    w = xf * att
    gids = lax.broadcasted_iota(jnp.int32, (1, G), 1)
    onehot = (batch_ref[...] == gids).astype(jnp.float32)
    pool = lax.dot_general(onehot, w, (((0,), (0,)), ((), ())),
                           preferred_element_type=jnp.float32,
                           precision=_HI)

    @pl.when(pl.program_id(0) == 0)
    def _():
        pooled_ref[...] = jnp.zeros_like(pooled_ref)
    pooled_ref[...] += pool


def _call_kc(outs, W_jump, b_jump, W_att, b_att, batch2):
    return pl.pallas_call(
        _kc_body,
        grid=(NRB,),
        in_specs=[pl.BlockSpec((RB, H), lambda i: (i, 0))] * 5 + [
            pl.BlockSpec((L * H, H), lambda i: (0, 0)),
            pl.BlockSpec((1, H), lambda i: (0, 0)),
            pl.BlockSpec((1, H), lambda i: (0, 0)),
            pl.BlockSpec((1, 1), lambda i: (0, 0)),
            pl.BlockSpec((RB, 1), lambda i: (i, 0)),
        ],
        out_specs=pl.BlockSpec((G, EMB), lambda i: (0, 0)),
        out_shape=jax.ShapeDtypeStruct((G, EMB), jnp.float32),
    )(*outs, W_jump, b_jump, W_att, b_att, batch2)


def _kd_body(pooled_ref, Wo1_ref, bo1_ref, Wo2_ref, bo2_ref, out_ref):
    t = jnp.dot(pooled_ref[...], Wo1_ref[...],
                preferred_element_type=jnp.float32, precision=_HI) + bo1_ref[...]
    t = jnp.maximum(t, 0.0)
    emb = jnp.dot(t, Wo2_ref[...], preferred_element_type=jnp.float32,
                  precision=_HI) + bo2_ref[...]
    nrm = jnp.maximum(jnp.sqrt(jnp.sum(emb * emb, axis=1, keepdims=True)),
                      1e-12)
    out_ref[...] = emb / nrm


def _call_kd(pooled, W_o1, b_o1, W_o2, b_o2):
    return pl.pallas_call(
        _kd_body,
        out_shape=jax.ShapeDtypeStruct((G, EMB), jnp.float32),
    )(pooled, W_o1, b_o1, W_o2, b_o2)


def kernel(x, edge_index, edge_attr, batch, W_in, b_in, bn_g, bn_b, eps,
           We, be, W1, b1, bn2_g, bn2_b, W2, b2, W_jump, b_jump, W_att,
           b_att, W_o1, b_o1, W_o2, b_o2):
    npad = E_PAD - E
    pad_ids = jnp.arange(npad, dtype=jnp.int32)
    src = jnp.concatenate([edge_index[0], pad_ids % N]).reshape(NW, EPW)
    dst = jnp.concatenate(
        [edge_index[1], N + pad_ids % (N_PAD - N)]).reshape(NW, NCHUNK, CH)
    ea_pad = jnp.concatenate(
        [edge_attr, jnp.zeros((npad, D_EDGE), jnp.float32)])
    batch2 = batch.reshape(N, 1)

    h, st_h = _call_k0(x, W_in, b_in.reshape(1, H))

    es = [_call_ke(ea_pad, We[l], be[l].reshape(1, H)).reshape(
        NW, NCHUNK, CH, H) for l in range(L)]

    outs = []
    for l in range(L):
        xn = _call_ka(h, st_h, bn_g[l].reshape(1, H), bn_b[l].reshape(1, H))
        parts = _sc_edge_aggr(xn, es[l], src, dst)[:, :N, :]
        t, st_t = _call_kb1(xn, parts, eps[l].reshape(1, 1), W1[l],
                            b1[l].reshape(1, 2 * H))
        h, st_h = _call_kb2(t, st_t, bn2_g[l].reshape(1, 2 * H),
                            bn2_b[l].reshape(1, 2 * H), W2[l],
                            b2[l].reshape(1, H), h)
        outs.append(h)

    pooled = _call_kc(outs, W_jump, b_jump.reshape(1, H),
                      W_att.reshape(1, H), b_att.reshape(1, 1), batch2)
    return _call_kd(pooled, W_o1, b_o1.reshape(1, EMB), W_o2,
                    b_o2.reshape(1, EMB))


# speedup vs baseline: 1.1851x; 1.0336x over previous
import functools
import math

import jax
import jax.numpy as jnp
from jax import lax
from jax.experimental import pallas as pl
from jax.experimental.pallas import tpu as pltpu
from jax.experimental.pallas import tpu_sc as plsc

N = 10000
E = 320000
D_IN = 128
D_EDGE = 16
H = 128
EMB = 128
L = 5
G = 64

_SC_INFO = plsc.get_sparse_core_info()
NC = _SC_INFO.num_cores
NS = _SC_INFO.num_subcores
NW = NC * NS
E_PAD = 322560
EPW = E_PAD // NW
CH = 48
NCHUNK = EPW // CH
N_PAD = 10112
RPT = N_PAD // NS

_SQRT2 = math.sqrt(2.0)
_HI = lax.Precision.HIGHEST


_sc_mesh = plsc.VectorSubcoreMesh(core_axis_name="c", subcore_axis_name="s")

NBUF = 3
NGROUP = NCHUNK // NBUF


@functools.partial(
    pl.kernel,
    out_type=jax.ShapeDtypeStruct((NC, N_PAD, H), jnp.float32),
    mesh=_sc_mesh,
    scratch_types=[
        pltpu.VMEM((EPW,), jnp.int32),
        pltpu.VMEM((NBUF, CH), jnp.int32),
        pltpu.VMEM((NBUF, CH, H), jnp.float32),
        pltpu.VMEM((NBUF, CH, H), jnp.float32),
        pltpu.VMEM_SHARED((N_PAD, H), jnp.float32),
    ] + [pltpu.SemaphoreType.DMA] * (2 * NBUF),
)
def _sc_edge_aggr(xn_hbm, e_hbm, src_hbm, dst_hbm, out_hbm,
                  srcv, dstv, msgv, xbuf, aggr_sh, *allsems):
    sems = allsems[:NBUF]
    ssems = allsems[NBUF:]
    cid = lax.axis_index("c")
    sid = lax.axis_index("s")
    wid = sid * NC + cid

    pltpu.sync_copy(src_hbm.at[wid], srcv)

    def _zrow(j, carry):
        for k in range(H // 16):
            xbuf[0, j, pl.ds(k * 16, 16)] = jnp.zeros((16,), jnp.float32)
        return carry
    lax.fori_loop(0, 64, _zrow, 0)
    for c in range(9):
        pltpu.sync_copy(xbuf.at[0, pl.ds(0, 64)],
                        aggr_sh.at[pl.ds(sid * RPT + c * 64, 64)])
    pltpu.sync_copy(xbuf.at[0, pl.ds(0, 56)],
                    aggr_sh.at[pl.ds(sid * RPT + 576, 56)])
    plsc.subcore_barrier()

    def _load(i, b):
        pltpu.async_copy(e_hbm.at[wid, i], msgv.at[b], sems[b])
        pltpu.async_copy(dst_hbm.at[wid, i], dstv.at[b], sems[b])
        pltpu.async_copy(xn_hbm.at[srcv.at[pl.ds(i * CH, CH)]], xbuf.at[b],
                         sems[b])

    def _start(i, b):
        pltpu.make_async_copy(msgv.at[b], aggr_sh.at[dstv.at[b]],
                              ssems[b]).wait()
        _load(i, b)

    def _finish(i, b):
        pltpu.make_async_copy(e_hbm.at[wid, i], msgv.at[b], sems[b]).wait()
        pltpu.make_async_copy(dst_hbm.at[wid, i], dstv.at[b], sems[b]).wait()
        pltpu.make_async_copy(xn_hbm.at[srcv.at[pl.ds(i * CH, CH)]],
                              xbuf.at[b], sems[b]).wait()

        @plsc.parallel_loop(0, CH, unroll=2)
        def _row(j):
            for k in range(H // 16):
                sl = pl.ds(k * 16, 16)
                msgv[b, j, sl] = jnp.maximum(msgv[b, j, sl] + xbuf[b, j, sl],
                                             0.0)
        pltpu.async_copy(msgv.at[b], aggr_sh.at[dstv.at[b]], ssems[b],
                         add=True)

    for b in range(NBUF):
        _load(b, b)

    def _group(g, carry):
        i0 = g * NBUF
        for b in range(NBUF):
            _finish(i0 + b, b)
            _start(i0 + b + NBUF, b)
        return carry
    lax.fori_loop(0, NGROUP - 1, _group, 0)
    for b in range(NBUF):
        _finish((NGROUP - 1) * NBUF + b, b)
    for b in range(NBUF):
        pltpu.make_async_copy(msgv.at[b], aggr_sh.at[dstv.at[b]],
                              ssems[b]).wait()

    plsc.subcore_barrier()
    pltpu.sync_copy(aggr_sh.at[pl.ds(sid * RPT, RPT)],
                    out_hbm.at[cid, pl.ds(sid * RPT, RPT)])


RB = 2000
NRB = N // RB


def _accum_stats(t, st_ref):
    @pl.when(pl.program_id(0) == 0)
    def _():
        st_ref[...] = jnp.zeros_like(st_ref)
    st_ref[...] += jnp.concatenate(
        [jnp.sum(t, axis=0, keepdims=True),
         jnp.sum(t * t, axis=0, keepdims=True)], axis=0)


def _mean_var(st_ref):
    m = st_ref[0:1, :] / N
    v = st_ref[1:2, :] / N - m * m
    return m, v


def _k0_body(x_ref, Wi_ref, bi_ref, h_ref, st_ref):
    h = jnp.dot(x_ref[...], Wi_ref[...], preferred_element_type=jnp.float32,
                precision=_HI) + bi_ref[...]
    h_ref[...] = h
    _accum_stats(h, st_ref)


def _call_k0(x, W_in, b_in):
    return pl.pallas_call(
        _k0_body,
        grid=(NRB,),
        in_specs=[
            pl.BlockSpec((RB, D_IN), lambda i: (i, 0)),
            pl.BlockSpec((D_IN, H), lambda i: (0, 0)),
            pl.BlockSpec((1, H), lambda i: (0, 0)),
        ],
        out_specs=(pl.BlockSpec((RB, H), lambda i: (i, 0)),
                   pl.BlockSpec((2, H), lambda i: (0, 0))),
        out_shape=(jax.ShapeDtypeStruct((N, H), jnp.float32),
                   jax.ShapeDtypeStruct((2, H), jnp.float32)),
    )(x, W_in, b_in)


def _ka_body(h_ref, st_ref, g_ref, b_ref, xn_ref):
    m, v = _mean_var(st_ref)
    xn_ref[...] = (g_ref[...] * (h_ref[...] - m) / jnp.sqrt(v + 1e-5)
                   + b_ref[...])


def _call_ka(h, st, g, b):
    return pl.pallas_call(
        _ka_body,
        grid=(NRB,),
        in_specs=[
            pl.BlockSpec((RB, H), lambda i: (i, 0)),
            pl.BlockSpec((2, H), lambda i: (0, 0)),
            pl.BlockSpec((1, H), lambda i: (0, 0)),
            pl.BlockSpec((1, H), lambda i: (0, 0)),
        ],
        out_specs=pl.BlockSpec((RB, H), lambda i: (i, 0)),
        out_shape=jax.ShapeDtypeStruct((N, H), jnp.float32),
    )(h, st, g, b)


_EBLK = 5040


def _ke_body(ea_ref, We_ref, be_ref, e_ref):
    e_ref[...] = jnp.dot(ea_ref[...], We_ref[...],
                         preferred_element_type=jnp.float32,
                         precision=_HI) + be_ref[...]


def _call_ke(edge_attr, We_l, be_l):
    return pl.pallas_call(
        _ke_body,
        grid=(E_PAD // _EBLK,),
        in_specs=[
            pl.BlockSpec((_EBLK, D_EDGE), lambda i: (i, 0)),
            pl.BlockSpec((D_EDGE, H), lambda i: (0, 0)),
            pl.BlockSpec((1, H), lambda i: (0, 0)),
        ],
        out_specs=pl.BlockSpec((_EBLK, H), lambda i: (i, 0)),
        out_shape=jax.ShapeDtypeStruct((E_PAD, H), jnp.float32),
    )(edge_attr, We_l, be_l)


def _kb1_body(xn_ref, p_ref, eps_ref, W1_ref, b1_ref, t_ref, st_ref):
    z = (1.0 + eps_ref[0, 0]) * xn_ref[...] + p_ref[0] + p_ref[1]
    t = jnp.dot(z, W1_ref[...], preferred_element_type=jnp.float32,
                precision=_HI) + b1_ref[...]
    t_ref[...] = t
    _accum_stats(t, st_ref)


def _call_kb1(xn, parts, eps_l, W1_l, b1_l):
    return pl.pallas_call(
        _kb1_body,
        grid=(NRB,),
        in_specs=[
            pl.BlockSpec((RB, H), lambda i: (i, 0)),
            pl.BlockSpec((2, RB, H), lambda i: (0, i, 0)),
            pl.BlockSpec((1, 1), lambda i: (0, 0)),
            pl.BlockSpec((H, 2 * H), lambda i: (0, 0)),
            pl.BlockSpec((1, 2 * H), lambda i: (0, 0)),
        ],
        out_specs=(pl.BlockSpec((RB, 2 * H), lambda i: (i, 0)),
                   pl.BlockSpec((2, 2 * H), lambda i: (0, 0))),
        out_shape=(jax.ShapeDtypeStruct((N, 2 * H), jnp.float32),
                   jax.ShapeDtypeStruct((2, 2 * H), jnp.float32)),
    )(xn, parts, eps_l, W1_l, b1_l)


def _kb2_body(t_ref, st_ref, g2_ref, b2_ref, W2_ref, bb_ref, h_ref,
              hn_ref, sth_ref):
    m, v = _mean_var(st_ref)
    tn = g2_ref[...] * (t_ref[...] - m) / jnp.sqrt(v + 1e-5) + b2_ref[...]
    tg = 0.5 * tn * (1.0 + lax.erf(tn / _SQRT2))
    z2 = jnp.dot(tg, W2_ref[...], preferred_element_type=jnp.float32,
                 precision=_HI) + bb_ref[...]
    hn = h_ref[...] + z2
    hn_ref[...] = hn
    _accum_stats(hn, sth_ref)


def _call_kb2(t, st, g2, b2, W2, bb, h):
    return pl.pallas_call(
        _kb2_body,
        grid=(NRB,),
        in_specs=[
            pl.BlockSpec((RB, 2 * H), lambda i: (i, 0)),
            pl.BlockSpec((2, 2 * H), lambda i: (0, 0)),
            pl.BlockSpec((1, 2 * H), lambda i: (0, 0)),
            pl.BlockSpec((1, 2 * H), lambda i: (0, 0)),
            pl.BlockSpec((2 * H, H), lambda i: (0, 0)),
            pl.BlockSpec((1, H), lambda i: (0, 0)),
            pl.BlockSpec((RB, H), lambda i: (i, 0)),
        ],
        out_specs=(pl.BlockSpec((RB, H), lambda i: (i, 0)),
                   pl.BlockSpec((2, H), lambda i: (0, 0))),
        out_shape=(jax.ShapeDtypeStruct((N, H), jnp.float32),
                   jax.ShapeDtypeStruct((2, H), jnp.float32)),
    )(t, st, g2, b2, W2, bb, h)


def _kc_body(o0, o1, o2, o3, o4, Wj_ref, bj_ref, Wa_ref, ba_ref,
             batch_ref, pooled_ref):
    xcat = jnp.concatenate([o0[...], o1[...], o2[...], o3[...], o4[...]],
                           axis=1)
    xf = jnp.dot(xcat, Wj_ref[...], preferred_element_type=jnp.float32,
                 precision=_HI) + bj_ref[...]
    logit = jnp.sum(xf * Wa_ref[...], axis=1, keepdims=True) + ba_ref[...]
    att = 1.0 / (1.0 + jnp.exp(-logit))
    w = xf * att
    gids = lax.broadcasted_iota(jnp.int32, (1, G), 1)
    onehot = (batch_ref[...] == gids).astype(jnp.float32)
    pool = lax.dot_general(onehot, w, (((0,), (0,)), ((), ())),
                           preferred_element_type=jnp.float32,
                           precision=_HI)

    @pl.when(pl.program_id(0) == 0)
    def _():
        pooled_ref[...] = jnp.zeros_like(pooled_ref)
    pooled_ref[...] += pool


def _call_kc(outs, W_jump, b_jump, W_att, b_att, batch2):
    return pl.pallas_call(
        _kc_body,
        grid=(NRB,),
        in_specs=[pl.BlockSpec((RB, H), lambda i: (i, 0))] * 5 + [
            pl.BlockSpec((L * H, H), lambda i: (0, 0)),
            pl.BlockSpec((1, H), lambda i: (0, 0)),
            pl.BlockSpec((1, H), lambda i: (0, 0)),
            pl.BlockSpec((1, 1), lambda i: (0, 0)),
            pl.BlockSpec((RB, 1), lambda i: (i, 0)),
        ],
        out_specs=pl.BlockSpec((G, EMB), lambda i: (0, 0)),
        out_shape=jax.ShapeDtypeStruct((G, EMB), jnp.float32),
    )(*outs, W_jump, b_jump, W_att, b_att, batch2)


def _kd_body(pooled_ref, Wo1_ref, bo1_ref, Wo2_ref, bo2_ref, out_ref):
    t = jnp.dot(pooled_ref[...], Wo1_ref[...],
                preferred_element_type=jnp.float32, precision=_HI) + bo1_ref[...]
    t = jnp.maximum(t, 0.0)
    emb = jnp.dot(t, Wo2_ref[...], preferred_element_type=jnp.float32,
                  precision=_HI) + bo2_ref[...]
    nrm = jnp.maximum(jnp.sqrt(jnp.sum(emb * emb, axis=1, keepdims=True)),
                      1e-12)
    out_ref[...] = emb / nrm


def _call_kd(pooled, W_o1, b_o1, W_o2, b_o2):
    return pl.pallas_call(
        _kd_body,
        out_shape=jax.ShapeDtypeStruct((G, EMB), jnp.float32),
    )(pooled, W_o1, b_o1, W_o2, b_o2)


def kernel(x, edge_index, edge_attr, batch, W_in, b_in, bn_g, bn_b, eps,
           We, be, W1, b1, bn2_g, bn2_b, W2, b2, W_jump, b_jump, W_att,
           b_att, W_o1, b_o1, W_o2, b_o2):
    npad = E_PAD - E
    pad_ids = jnp.arange(npad, dtype=jnp.int32)
    src = jnp.concatenate([edge_index[0], pad_ids % N]).reshape(NW, EPW)
    dst = jnp.concatenate(
        [edge_index[1], N + pad_ids % (N_PAD - N)]).reshape(NW, NCHUNK, CH)
    ea_pad = jnp.concatenate(
        [edge_attr, jnp.zeros((npad, D_EDGE), jnp.float32)])
    batch2 = batch.reshape(N, 1)

    h, st_h = _call_k0(x, W_in, b_in.reshape(1, H))

    es = [_call_ke(ea_pad, We[l], be[l].reshape(1, H)).reshape(
        NW, NCHUNK, CH, H) for l in range(L)]

    outs = []
    for l in range(L):
        xn = _call_ka(h, st_h, bn_g[l].reshape(1, H), bn_b[l].reshape(1, H))
        parts = _sc_edge_aggr(xn, es[l], src, dst)[:, :N, :]
        t, st_t = _call_kb1(xn, parts, eps[l].reshape(1, 1), W1[l],
                            b1[l].reshape(1, 2 * H))
        h, st_h = _call_kb2(t, st_t, bn2_g[l].reshape(1, 2 * H),
                            bn2_b[l].reshape(1, 2 * H), W2[l],
                            b2[l].reshape(1, H), h)
        outs.append(h)

    pooled = _call_kc(outs, W_jump, b_jump.reshape(1, H),
                      W_att.reshape(1, H), b_att.reshape(1, 1), batch2)
    return _call_kd(pooled, W_o1, b_o1.reshape(1, EMB), W_o2,
                    b_o2.reshape(1, EMB))


# speedup vs baseline: 1.1910x; 1.0050x over previous
import functools
import math

import jax
import jax.numpy as jnp
from jax import lax
from jax.experimental import pallas as pl
from jax.experimental.pallas import tpu as pltpu
from jax.experimental.pallas import tpu_sc as plsc

N = 10000
E = 320000
D_IN = 128
D_EDGE = 16
H = 128
EMB = 128
L = 5
G = 64

_SC_INFO = plsc.get_sparse_core_info()
NC = _SC_INFO.num_cores
NS = _SC_INFO.num_subcores
NW = NC * NS
E_PAD = 322560
EPW = E_PAD // NW
CH = 48
NCHUNK = EPW // CH
N_PAD = 10112
RPT = N_PAD // NS

_SQRT2 = math.sqrt(2.0)
_HI = lax.Precision.HIGHEST


_sc_mesh = plsc.VectorSubcoreMesh(core_axis_name="c", subcore_axis_name="s")

NBUF = 3
NGROUP = NCHUNK // NBUF


@functools.partial(
    pl.kernel,
    out_type=jax.ShapeDtypeStruct((NC, N_PAD, H), jnp.float32),
    mesh=_sc_mesh,
    scratch_types=[
        pltpu.VMEM((EPW,), jnp.int32),
        pltpu.VMEM((NBUF, CH), jnp.int32),
        pltpu.VMEM((NBUF, CH, H), jnp.float32),
        pltpu.VMEM((NBUF, CH, H), jnp.float32),
        pltpu.VMEM_SHARED((N_PAD, H), jnp.float32),
    ] + [pltpu.SemaphoreType.DMA] * (2 * NBUF),
)
def _sc_edge_aggr(xn_hbm, e_hbm, src_hbm, dst_hbm, out_hbm,
                  srcv, dstv, msgv, xbuf, aggr_sh, *allsems):
    sems = allsems[:NBUF]
    ssems = allsems[NBUF:]
    cid = lax.axis_index("c")
    sid = lax.axis_index("s")
    wid = sid * NC + cid

    pltpu.async_copy(src_hbm.at[wid], srcv, sems[0])

    def _zrow(j, carry):
        for k in range(H // 16):
            xbuf[0, j, pl.ds(k * 16, 16)] = jnp.zeros((16,), jnp.float32)
        return carry
    lax.fori_loop(0, 64, _zrow, 0)
    for c in range(9):
        pltpu.async_copy(xbuf.at[0, pl.ds(0, 64)],
                         aggr_sh.at[pl.ds(sid * RPT + c * 64, 64)], sems[1])
    pltpu.async_copy(xbuf.at[0, pl.ds(0, 56)],
                     aggr_sh.at[pl.ds(sid * RPT + 576, 56)], sems[1])
    pltpu.make_async_copy(src_hbm.at[wid], srcv, sems[0]).wait()
    for c in range(9):
        pltpu.make_async_copy(xbuf.at[0, pl.ds(0, 64)],
                              aggr_sh.at[pl.ds(sid * RPT + c * 64, 64)],
                              sems[1]).wait()
    pltpu.make_async_copy(xbuf.at[0, pl.ds(0, 56)],
                          aggr_sh.at[pl.ds(sid * RPT + 576, 56)],
                          sems[1]).wait()
    plsc.subcore_barrier()

    def _load(i, b):
        pltpu.async_copy(e_hbm.at[wid, i], msgv.at[b], sems[b])
        pltpu.async_copy(dst_hbm.at[wid, i], dstv.at[b], sems[b])
        pltpu.async_copy(xn_hbm.at[srcv.at[pl.ds(i * CH, CH)]], xbuf.at[b],
                         sems[b])

    def _start(i, b):
        pltpu.make_async_copy(msgv.at[b], aggr_sh.at[dstv.at[b]],
                              ssems[b]).wait()
        _load(i, b)

    def _finish(i, b):
        pltpu.make_async_copy(e_hbm.at[wid, i], msgv.at[b], sems[b]).wait()
        pltpu.make_async_copy(dst_hbm.at[wid, i], dstv.at[b], sems[b]).wait()
        pltpu.make_async_copy(xn_hbm.at[srcv.at[pl.ds(i * CH, CH)]],
                              xbuf.at[b], sems[b]).wait()

        @plsc.parallel_loop(0, CH, unroll=2)
        def _row(j):
            for k in range(H // 16):
                sl = pl.ds(k * 16, 16)
                msgv[b, j, sl] = jnp.maximum(msgv[b, j, sl] + xbuf[b, j, sl],
                                             0.0)
        pltpu.async_copy(msgv.at[b], aggr_sh.at[dstv.at[b]], ssems[b],
                         add=True)

    for b in range(NBUF):
        _load(b, b)

    def _group(g, carry):
        i0 = g * NBUF
        for b in range(NBUF):
            _finish(i0 + b, b)
            _start(i0 + b + NBUF, b)
        return carry
    lax.fori_loop(0, NGROUP - 1, _group, 0)
    for b in range(NBUF):
        _finish((NGROUP - 1) * NBUF + b, b)
    for b in range(NBUF):
        pltpu.make_async_copy(msgv.at[b], aggr_sh.at[dstv.at[b]],
                              ssems[b]).wait()

    plsc.subcore_barrier()
    pltpu.sync_copy(aggr_sh.at[pl.ds(sid * RPT, RPT)],
                    out_hbm.at[cid, pl.ds(sid * RPT, RPT)])


RB = 2000
NRB = N // RB


def _accum_stats(t, st_ref):
    @pl.when(pl.program_id(0) == 0)
    def _():
        st_ref[...] = jnp.zeros_like(st_ref)
    st_ref[...] += jnp.concatenate(
        [jnp.sum(t, axis=0, keepdims=True),
         jnp.sum(t * t, axis=0, keepdims=True)], axis=0)


def _mean_var(st_ref):
    m = st_ref[0:1, :] / N
    v = st_ref[1:2, :] / N - m * m
    return m, v


def _k0_body(x_ref, Wi_ref, bi_ref, h_ref, st_ref):
    h = jnp.dot(x_ref[...], Wi_ref[...], preferred_element_type=jnp.float32,
                precision=_HI) + bi_ref[...]
    h_ref[...] = h
    _accum_stats(h, st_ref)


def _call_k0(x, W_in, b_in):
    return pl.pallas_call(
        _k0_body,
        grid=(NRB,),
        in_specs=[
            pl.BlockSpec((RB, D_IN), lambda i: (i, 0)),
            pl.BlockSpec((D_IN, H), lambda i: (0, 0)),
            pl.BlockSpec((1, H), lambda i: (0, 0)),
        ],
        out_specs=(pl.BlockSpec((RB, H), lambda i: (i, 0)),
                   pl.BlockSpec((2, H), lambda i: (0, 0))),
        out_shape=(jax.ShapeDtypeStruct((N, H), jnp.float32),
                   jax.ShapeDtypeStruct((2, H), jnp.float32)),
    )(x, W_in, b_in)


def _ka_body(h_ref, st_ref, g_ref, b_ref, xn_ref):
    m, v = _mean_var(st_ref)
    xn_ref[...] = (g_ref[...] * (h_ref[...] - m) / jnp.sqrt(v + 1e-5)
                   + b_ref[...])


def _call_ka(h, st, g, b):
    return pl.pallas_call(
        _ka_body,
        grid=(NRB,),
        in_specs=[
            pl.BlockSpec((RB, H), lambda i: (i, 0)),
            pl.BlockSpec((2, H), lambda i: (0, 0)),
            pl.BlockSpec((1, H), lambda i: (0, 0)),
            pl.BlockSpec((1, H), lambda i: (0, 0)),
        ],
        out_specs=pl.BlockSpec((RB, H), lambda i: (i, 0)),
        out_shape=jax.ShapeDtypeStruct((N, H), jnp.float32),
    )(h, st, g, b)


_EBLK = 5040


def _ke_body(ea_ref, We_ref, be_ref, e_ref):
    e_ref[...] = jnp.dot(ea_ref[...], We_ref[...],
                         preferred_element_type=jnp.float32,
                         precision=_HI) + be_ref[...]


def _call_ke(edge_attr, We_l, be_l):
    return pl.pallas_call(
        _ke_body,
        grid=(E_PAD // _EBLK,),
        in_specs=[
            pl.BlockSpec((_EBLK, D_EDGE), lambda i: (i, 0)),
            pl.BlockSpec((D_EDGE, H), lambda i: (0, 0)),
            pl.BlockSpec((1, H), lambda i: (0, 0)),
        ],
        out_specs=pl.BlockSpec((_EBLK, H), lambda i: (i, 0)),
        out_shape=jax.ShapeDtypeStruct((E_PAD, H), jnp.float32),
    )(edge_attr, We_l, be_l)


def _kb1_body(xn_ref, p_ref, eps_ref, W1_ref, b1_ref, t_ref, st_ref):
    z = (1.0 + eps_ref[0, 0]) * xn_ref[...] + p_ref[0] + p_ref[1]
    t = jnp.dot(z, W1_ref[...], preferred_element_type=jnp.float32,
                precision=_HI) + b1_ref[...]
    t_ref[...] = t
    _accum_stats(t, st_ref)


def _call_kb1(xn, parts, eps_l, W1_l, b1_l):
    return pl.pallas_call(
        _kb1_body,
        grid=(NRB,),
        in_specs=[
            pl.BlockSpec((RB, H), lambda i: (i, 0)),
            pl.BlockSpec((2, RB, H), lambda i: (0, i, 0)),
            pl.BlockSpec((1, 1), lambda i: (0, 0)),
            pl.BlockSpec((H, 2 * H), lambda i: (0, 0)),
            pl.BlockSpec((1, 2 * H), lambda i: (0, 0)),
        ],
        out_specs=(pl.BlockSpec((RB, 2 * H), lambda i: (i, 0)),
                   pl.BlockSpec((2, 2 * H), lambda i: (0, 0))),
        out_shape=(jax.ShapeDtypeStruct((N, 2 * H), jnp.float32),
                   jax.ShapeDtypeStruct((2, 2 * H), jnp.float32)),
    )(xn, parts, eps_l, W1_l, b1_l)


def _kb2_body(t_ref, st_ref, g2_ref, b2_ref, W2_ref, bb_ref, h_ref,
              hn_ref, sth_ref):
    m, v = _mean_var(st_ref)
    tn = g2_ref[...] * (t_ref[...] - m) / jnp.sqrt(v + 1e-5) + b2_ref[...]
    tg = 0.5 * tn * (1.0 + lax.erf(tn / _SQRT2))
    z2 = jnp.dot(tg, W2_ref[...], preferred_element_type=jnp.float32,
                 precision=_HI) + bb_ref[...]
    hn = h_ref[...] + z2
    hn_ref[...] = hn
    _accum_stats(hn, sth_ref)


def _call_kb2(t, st, g2, b2, W2, bb, h):
    return pl.pallas_call(
        _kb2_body,
        grid=(NRB,),
        in_specs=[
            pl.BlockSpec((RB, 2 * H), lambda i: (i, 0)),
            pl.BlockSpec((2, 2 * H), lambda i: (0, 0)),
            pl.BlockSpec((1, 2 * H), lambda i: (0, 0)),
            pl.BlockSpec((1, 2 * H), lambda i: (0, 0)),
            pl.BlockSpec((2 * H, H), lambda i: (0, 0)),
            pl.BlockSpec((1, H), lambda i: (0, 0)),
            pl.BlockSpec((RB, H), lambda i: (i, 0)),
        ],
        out_specs=(pl.BlockSpec((RB, H), lambda i: (i, 0)),
                   pl.BlockSpec((2, H), lambda i: (0, 0))),
        out_shape=(jax.ShapeDtypeStruct((N, H), jnp.float32),
                   jax.ShapeDtypeStruct((2, H), jnp.float32)),
    )(t, st, g2, b2, W2, bb, h)


def _kc_body(o0, o1, o2, o3, o4, Wj_ref, bj_ref, Wa_ref, ba_ref,
             batch_ref, pooled_ref):
    xcat = jnp.concatenate([o0[...], o1[...], o2[...], o3[...], o4[...]],
                           axis=1)
    xf = jnp.dot(xcat, Wj_ref[...], preferred_element_type=jnp.float32,
                 precision=_HI) + bj_ref[...]
    logit = jnp.sum(xf * Wa_ref[...], axis=1, keepdims=True) + ba_ref[...]
    att = 1.0 / (1.0 + jnp.exp(-logit))
    w = xf * att
    gids = lax.broadcasted_iota(jnp.int32, (1, G), 1)
    onehot = (batch_ref[...] == gids).astype(jnp.float32)
    pool = lax.dot_general(onehot, w, (((0,), (0,)), ((), ())),
                           preferred_element_type=jnp.float32,
                           precision=_HI)

    @pl.when(pl.program_id(0) == 0)
    def _():
        pooled_ref[...] = jnp.zeros_like(pooled_ref)
    pooled_ref[...] += pool


def _call_kc(outs, W_jump, b_jump, W_att, b_att, batch2):
    return pl.pallas_call(
        _kc_body,
        grid=(NRB,),
        in_specs=[pl.BlockSpec((RB, H), lambda i: (i, 0))] * 5 + [
            pl.BlockSpec((L * H, H), lambda i: (0, 0)),
            pl.BlockSpec((1, H), lambda i: (0, 0)),
            pl.BlockSpec((1, H), lambda i: (0, 0)),
            pl.BlockSpec((1, 1), lambda i: (0, 0)),
            pl.BlockSpec((RB, 1), lambda i: (i, 0)),
        ],
        out_specs=pl.BlockSpec((G, EMB), lambda i: (0, 0)),
        out_shape=jax.ShapeDtypeStruct((G, EMB), jnp.float32),
    )(*outs, W_jump, b_jump, W_att, b_att, batch2)


def _kd_body(pooled_ref, Wo1_ref, bo1_ref, Wo2_ref, bo2_ref, out_ref):
    t = jnp.dot(pooled_ref[...], Wo1_ref[...],
                preferred_element_type=jnp.float32, precision=_HI) + bo1_ref[...]
    t = jnp.maximum(t, 0.0)
    emb = jnp.dot(t, Wo2_ref[...], preferred_element_type=jnp.float32,
                  precision=_HI) + bo2_ref[...]
    nrm = jnp.maximum(jnp.sqrt(jnp.sum(emb * emb, axis=1, keepdims=True)),
                      1e-12)
    out_ref[...] = emb / nrm


def _call_kd(pooled, W_o1, b_o1, W_o2, b_o2):
    return pl.pallas_call(
        _kd_body,
        out_shape=jax.ShapeDtypeStruct((G, EMB), jnp.float32),
    )(pooled, W_o1, b_o1, W_o2, b_o2)


def kernel(x, edge_index, edge_attr, batch, W_in, b_in, bn_g, bn_b, eps,
           We, be, W1, b1, bn2_g, bn2_b, W2, b2, W_jump, b_jump, W_att,
           b_att, W_o1, b_o1, W_o2, b_o2):
    npad = E_PAD - E
    pad_ids = jnp.arange(npad, dtype=jnp.int32)
    src = jnp.concatenate([edge_index[0], pad_ids % N]).reshape(NW, EPW)
    dst = jnp.concatenate(
        [edge_index[1], N + pad_ids % (N_PAD - N)]).reshape(NW, NCHUNK, CH)
    ea_pad = jnp.concatenate(
        [edge_attr, jnp.zeros((npad, D_EDGE), jnp.float32)])
    batch2 = batch.reshape(N, 1)

    h, st_h = _call_k0(x, W_in, b_in.reshape(1, H))

    es = [_call_ke(ea_pad, We[l], be[l].reshape(1, H)).reshape(
        NW, NCHUNK, CH, H) for l in range(L)]

    outs = []
    for l in range(L):
        xn = _call_ka(h, st_h, bn_g[l].reshape(1, H), bn_b[l].reshape(1, H))
        parts = _sc_edge_aggr(xn, es[l], src, dst)[:, :N, :]
        t, st_t = _call_kb1(xn, parts, eps[l].reshape(1, 1), W1[l],
                            b1[l].reshape(1, 2 * H))
        h, st_h = _call_kb2(t, st_t, bn2_g[l].reshape(1, 2 * H),
                            bn2_b[l].reshape(1, 2 * H), W2[l],
                            b2[l].reshape(1, H), h)
        outs.append(h)

    pooled = _call_kc(outs, W_jump, b_jump.reshape(1, H),
                      W_att.reshape(1, H), b_att.reshape(1, 1), batch2)
    return _call_kd(pooled, W_o1, b_o1.reshape(1, EMB), W_o2,
                    b_o2.reshape(1, EMB))
